# baseline (speedup 1.0000x reference)
import functools

import jax
import jax.numpy as jnp
from jax import lax
from jax.experimental import pallas as pl
from jax.experimental.pallas import tpu as pltpu
from jax.experimental.pallas import tpu_sc as plsc

D_MODEL = 64
D_PAD = 128
SCALE = float(D_MODEL) ** 0.5
NUM_CORES = 2
NUM_SUBCORES = 16
NUM_WORKERS = NUM_CORES * NUM_SUBCORES
CHUNK = 640
NBUF = 2
NIBUF = 3


@functools.partial(jax.jit, static_argnums=(2,))
def _sc_embed(idx, table_padded, n_rows):
    b_per_w = n_rows // NUM_WORKERS
    n_chunks = b_per_w // CHUNK
    mesh = plsc.VectorSubcoreMesh(core_axis_name="c", subcore_axis_name="s")

    @functools.partial(
        pl.kernel,
        mesh=mesh,
        out_type=jax.ShapeDtypeStruct((n_rows, D_PAD), jnp.float32),
        scratch_types=[
            [pltpu.VMEM((CHUNK,), jnp.int32) for _ in range(NIBUF)],
            [pltpu.VMEM((CHUNK, D_MODEL), jnp.float32) for _ in range(NBUF)],
            [pltpu.SemaphoreType.DMA for _ in range(NIBUF)],
            [pltpu.SemaphoreType.DMA for _ in range(NBUF)],
            [pltpu.SemaphoreType.DMA for _ in range(NBUF)],
        ],
        compiler_params=pltpu.CompilerParams(use_tc_tiling_on_sc=False),
    )
    def k(idx_hbm, table_hbm, out_hbm, idxs, rows, i_sems, g_sems, w_sems):
        wid = lax.axis_index("s") * NUM_CORES + lax.axis_index("c")
        base_w = wid * b_per_w

        def start_idx(c):
            b = c % NIBUF
            return pltpu.async_copy(
                idx_hbm.at[pl.ds(base_w + c * CHUNK, CHUNK)], idxs[b], i_sems[b]
            )

        def start_gather(c):
            return pltpu.async_copy(
                table_hbm.at[idxs[c % NIBUF]], rows[c % NBUF], g_sems[c % NBUF]
            )

        idx_copies = {c: start_idx(c) for c in range(min(2, n_chunks))}
        idx_copies.pop(0).wait()
        gathers = {0: start_gather(0)}
        writebacks = {}
        for c in range(n_chunks):
            gathers.pop(c).wait()
            if c + 1 < n_chunks:
                if c + 2 < n_chunks:
                    idx_copies[c + 2] = start_idx(c + 2)
                idx_copies.pop(c + 1).wait()
                if c + 1 >= NBUF:
                    writebacks.pop(c + 1 - NBUF).wait()
                gathers[c + 1] = start_gather(c + 1)

            buf = rows[c % NBUF]

            @plsc.parallel_loop(0, CHUNK, step=1, unroll=8)
            def _scale(r):
                for j in range(D_MODEL // 16):
                    sl = (r, pl.ds(j * 16, 16))
                    buf[sl] = buf[sl] * SCALE

            writebacks[c] = pltpu.async_copy(
                buf,
                out_hbm.at[pl.ds(base_w + c * CHUNK, CHUNK), pl.ds(0, D_MODEL)],
                w_sems[c % NBUF],
            )
        for c in sorted(writebacks):
            writebacks.pop(c).wait()

    return k(idx, table_padded)


def kernel(x, table):
    b, s = x.shape
    n = b * s
    v = table.shape[0]
    xf = x.reshape(n).astype(jnp.int32)
    th = jnp.concatenate([table[0::2], table[1::2]], axis=1)
    th = lax.optimization_barrier(th).reshape(v, D_MODEL)
    out = _sc_embed(xf, th, n)
    return out.reshape(b, s, D_PAD)[:, :, :D_MODEL]


# speedup vs baseline: 24.7001x; 24.7001x over previous
import functools

import jax
import jax.numpy as jnp
from jax import lax
from jax.experimental import pallas as pl
from jax.experimental.pallas import tpu as pltpu
from jax.experimental.pallas import tpu_sc as plsc

D_MODEL = 64
D_PAD = 128
SCALE = float(D_MODEL) ** 0.5
NUM_CORES = 2
NUM_SUBCORES = 16
NUM_WORKERS = NUM_CORES * NUM_SUBCORES
CHUNK = 640
NBUF = 2
NIBUF = 3
FMT_BLK = 8192


def _tc_format_body(t_ref, o_ref):
    t = jnp.swapaxes(t_ref[...], 0, 1) * SCALE
    o_ref[:, :D_MODEL] = t[: FMT_BLK // 2]
    o_ref[:, D_MODEL:] = t[FMT_BLK // 2 :]


def _tc_format(tt):
    v = tt.shape[1]
    n_blk = (v + FMT_BLK - 1) // FMT_BLK
    return pl.pallas_call(
        _tc_format_body,
        grid=(n_blk,),
        in_specs=[pl.BlockSpec((D_MODEL, FMT_BLK), lambda i: (0, i))],
        out_specs=pl.BlockSpec((FMT_BLK // 2, 2 * D_MODEL), lambda i: (i, 0)),
        out_shape=jax.ShapeDtypeStruct((n_blk * FMT_BLK // 2, 2 * D_MODEL), jnp.float32),
    )(tt)


@functools.partial(jax.jit, static_argnums=(2,))
def _sc_embed(idx, table_scaled, n_rows):
    b_per_w = n_rows // NUM_WORKERS
    n_chunks = b_per_w // CHUNK
    mesh = plsc.VectorSubcoreMesh(core_axis_name="c", subcore_axis_name="s")

    @functools.partial(
        pl.kernel,
        mesh=mesh,
        out_type=jax.ShapeDtypeStruct((n_rows, D_PAD), jnp.float32),
        scratch_types=[
            [pltpu.VMEM((CHUNK,), jnp.int32) for _ in range(NIBUF)],
            [pltpu.VMEM((CHUNK, D_MODEL), jnp.float32) for _ in range(NBUF)],
            [pltpu.SemaphoreType.DMA for _ in range(NIBUF)],
            [pltpu.SemaphoreType.DMA for _ in range(NBUF)],
            [pltpu.SemaphoreType.DMA for _ in range(NBUF)],
        ],
        compiler_params=pltpu.CompilerParams(use_tc_tiling_on_sc=False),
    )
    def k(idx_hbm, table_hbm, out_hbm, idxs, rows, i_sems, g_sems, w_sems):
        wid = lax.axis_index("s") * NUM_CORES + lax.axis_index("c")
        base_w = wid * b_per_w

        def start_idx(c):
            b = c % NIBUF
            return pltpu.async_copy(
                idx_hbm.at[pl.ds(base_w + c * CHUNK, CHUNK)], idxs[b], i_sems[b]
            )

        def start_gather(c):
            return pltpu.async_copy(
                table_hbm.at[idxs[c % NIBUF]], rows[c % NBUF], g_sems[c % NBUF]
            )

        idx_copies = {c: start_idx(c) for c in range(min(2, n_chunks))}
        idx_copies.pop(0).wait()
        gathers = {0: start_gather(0)}
        writebacks = {}
        for c in range(n_chunks):
            gathers.pop(c).wait()
            if c + 1 < n_chunks:
                if c + 2 < n_chunks:
                    idx_copies[c + 2] = start_idx(c + 2)
                idx_copies.pop(c + 1).wait()
                if c + 1 >= NBUF:
                    writebacks.pop(c + 1 - NBUF).wait()
                gathers[c + 1] = start_gather(c + 1)

            writebacks[c] = pltpu.async_copy(
                rows[c % NBUF],
                out_hbm.at[pl.ds(base_w + c * CHUNK, CHUNK), pl.ds(0, D_MODEL)],
                w_sems[c % NBUF],
            )
        for c in sorted(writebacks):
            writebacks.pop(c).wait()

    return k(idx, table_scaled)


def kernel(x, table):
    b, s = x.shape
    n = b * s
    v = table.shape[0]
    xf = x.reshape(n).astype(jnp.int32)
    blk = xf >> 13
    m = xf & (FMT_BLK - 1)
    xg = (blk << 13) + ((m & (FMT_BLK // 2 - 1)) << 1) + (m >> 12)
    tf = _tc_format(table.T)
    th = tf.reshape(tf.shape[0] * 2, D_MODEL)
    out = _sc_embed(xg, th, n)
    return out.reshape(b, s, D_PAD)[:, :, :D_MODEL]


# speedup vs baseline: 27.0615x; 1.0956x over previous
import functools

import jax
import jax.numpy as jnp
from jax import lax
from jax.experimental import pallas as pl
from jax.experimental.pallas import tpu as pltpu
from jax.experimental.pallas import tpu_sc as plsc

D_MODEL = 64
D_PAD = 128
SCALE = float(D_MODEL) ** 0.5
NUM_CORES = 2
NUM_SUBCORES = 16
NUM_WORKERS = NUM_CORES * NUM_SUBCORES
CHUNK = 640
NBUF = 2
NIBUF = 3
FMT_BLK = 16384


def _tc_format_body(t_ref, o_ref):
    t = jnp.swapaxes(t_ref[...], 0, 1) * SCALE
    o_ref[:, :D_MODEL] = t[: FMT_BLK // 2]
    o_ref[:, D_MODEL:] = t[FMT_BLK // 2 :]


def _tc_format(tt):
    v = tt.shape[1]
    n_blk = (v + FMT_BLK - 1) // FMT_BLK
    return pl.pallas_call(
        _tc_format_body,
        grid=(n_blk,),
        in_specs=[pl.BlockSpec((D_MODEL, FMT_BLK), lambda i: (0, i))],
        out_specs=pl.BlockSpec((FMT_BLK // 2, 2 * D_MODEL), lambda i: (i, 0)),
        out_shape=jax.ShapeDtypeStruct((n_blk * FMT_BLK // 2, 2 * D_MODEL), jnp.float32),
    )(tt)


@functools.partial(jax.jit, static_argnums=(2,))
def _sc_embed(idx, table_scaled, n_rows):
    b_per_w = n_rows // NUM_WORKERS
    n_chunks = b_per_w // CHUNK
    mesh = plsc.VectorSubcoreMesh(core_axis_name="c", subcore_axis_name="s")

    @functools.partial(
        pl.kernel,
        mesh=mesh,
        out_type=jax.ShapeDtypeStruct((n_rows, D_PAD), jnp.float32),
        scratch_types=[
            [pltpu.VMEM((CHUNK,), jnp.int32) for _ in range(NIBUF)],
            [pltpu.VMEM((CHUNK, D_MODEL), jnp.float32) for _ in range(NBUF)],
            [pltpu.SemaphoreType.DMA for _ in range(NIBUF)],
            [pltpu.SemaphoreType.DMA for _ in range(NBUF)],
            [pltpu.SemaphoreType.DMA for _ in range(NBUF)],
        ],
        compiler_params=pltpu.CompilerParams(use_tc_tiling_on_sc=False),
    )
    def k(idx_hbm, table_hbm, out_hbm, idxs, rows, i_sems, g_sems, w_sems):
        wid = lax.axis_index("s") * NUM_CORES + lax.axis_index("c")
        base_w = wid * b_per_w

        def start_idx(c):
            b = c % NIBUF
            return pltpu.async_copy(
                idx_hbm.at[pl.ds(base_w + c * CHUNK, CHUNK)], idxs[b], i_sems[b]
            )

        def start_gather(c):
            return pltpu.async_copy(
                table_hbm.at[idxs[c % NIBUF]], rows[c % NBUF], g_sems[c % NBUF]
            )

        idx_copies = {c: start_idx(c) for c in range(min(2, n_chunks))}
        idx_copies.pop(0).wait()
        gathers = {0: start_gather(0)}
        writebacks = {}
        for c in range(n_chunks):
            gathers.pop(c).wait()
            if c + 1 < n_chunks:
                if c + 2 < n_chunks:
                    idx_copies[c + 2] = start_idx(c + 2)
                idx_copies.pop(c + 1).wait()
                if c + 1 >= NBUF:
                    writebacks.pop(c + 1 - NBUF).wait()
                gathers[c + 1] = start_gather(c + 1)

            writebacks[c] = pltpu.async_copy(
                rows[c % NBUF],
                out_hbm.at[pl.ds(base_w + c * CHUNK, CHUNK), pl.ds(0, D_MODEL)],
                w_sems[c % NBUF],
            )
        for c in sorted(writebacks):
            writebacks.pop(c).wait()

    return k(idx, table_scaled)


def kernel(x, table):
    b, s = x.shape
    n = b * s
    v = table.shape[0]
    xf = x.reshape(n).astype(jnp.int32)
    blk = xf >> 14
    m = xf & (FMT_BLK - 1)
    xg = (blk << 14) + ((m & (FMT_BLK // 2 - 1)) << 1) + (m >> 13)
    tf = _tc_format(table.T)
    th = tf.reshape(tf.shape[0] * 2, D_MODEL)
    out = _sc_embed(xg, th, n)
    return out.reshape(b, s, D_PAD)[:, :, :D_MODEL]


# speedup vs baseline: 28.3700x; 1.0484x over previous
import functools

import jax
import jax.numpy as jnp
from jax import lax
from jax.experimental import pallas as pl
from jax.experimental.pallas import tpu as pltpu
from jax.experimental.pallas import tpu_sc as plsc

D_MODEL = 64
D_PAD = 128
SCALE = float(D_MODEL) ** 0.5
NUM_CORES = 2
NUM_SUBCORES = 16
NUM_WORKERS = NUM_CORES * NUM_SUBCORES
CHUNK = 640
NBUF = 2
NIBUF = 3
FMT_BLK = 32768


def _tc_format_body(t_ref, o_ref):
    t = jnp.swapaxes(t_ref[...], 0, 1) * SCALE
    o_ref[:, :D_MODEL] = t[: FMT_BLK // 2]
    o_ref[:, D_MODEL:] = t[FMT_BLK // 2 :]


def _tc_format(tt):
    v = tt.shape[1]
    n_blk = (v + FMT_BLK - 1) // FMT_BLK
    return pl.pallas_call(
        _tc_format_body,
        grid=(n_blk,),
        in_specs=[pl.BlockSpec((D_MODEL, FMT_BLK), lambda i: (0, i))],
        out_specs=pl.BlockSpec((FMT_BLK // 2, 2 * D_MODEL), lambda i: (i, 0)),
        out_shape=jax.ShapeDtypeStruct((n_blk * FMT_BLK // 2, 2 * D_MODEL), jnp.float32),
    )(tt)


@functools.partial(jax.jit, static_argnums=(2,))
def _sc_embed(idx, table_scaled, n_rows):
    b_per_w = n_rows // NUM_WORKERS
    n_chunks = b_per_w // CHUNK
    mesh = plsc.VectorSubcoreMesh(core_axis_name="c", subcore_axis_name="s")

    @functools.partial(
        pl.kernel,
        mesh=mesh,
        out_type=jax.ShapeDtypeStruct((n_rows, D_PAD), jnp.float32),
        scratch_types=[
            [pltpu.VMEM((CHUNK,), jnp.int32) for _ in range(NIBUF)],
            [pltpu.VMEM((CHUNK, D_MODEL), jnp.float32) for _ in range(NBUF)],
            [pltpu.SemaphoreType.DMA for _ in range(NIBUF)],
            [pltpu.SemaphoreType.DMA for _ in range(NBUF)],
            [pltpu.SemaphoreType.DMA for _ in range(NBUF)],
        ],
        compiler_params=pltpu.CompilerParams(use_tc_tiling_on_sc=False),
    )
    def k(idx_hbm, table_hbm, out_hbm, idxs, rows, i_sems, g_sems, w_sems):
        wid = lax.axis_index("s") * NUM_CORES + lax.axis_index("c")
        base_w = wid * b_per_w

        def start_idx(c):
            b = c % NIBUF
            return pltpu.async_copy(
                idx_hbm.at[pl.ds(base_w + c * CHUNK, CHUNK)], idxs[b], i_sems[b]
            )

        def start_gather(c):
            return pltpu.async_copy(
                table_hbm.at[idxs[c % NIBUF]], rows[c % NBUF], g_sems[c % NBUF]
            )

        idx_copies = {c: start_idx(c) for c in range(min(2, n_chunks))}
        idx_copies.pop(0).wait()
        gathers = {0: start_gather(0)}
        writebacks = {}
        for c in range(n_chunks):
            gathers.pop(c).wait()
            if c + 1 < n_chunks:
                if c + 2 < n_chunks:
                    idx_copies[c + 2] = start_idx(c + 2)
                idx_copies.pop(c + 1).wait()
                if c + 1 >= NBUF:
                    writebacks.pop(c + 1 - NBUF).wait()
                gathers[c + 1] = start_gather(c + 1)

            writebacks[c] = pltpu.async_copy(
                rows[c % NBUF],
                out_hbm.at[pl.ds(base_w + c * CHUNK, CHUNK), pl.ds(0, D_MODEL)],
                w_sems[c % NBUF],
            )
        for c in sorted(writebacks):
            writebacks.pop(c).wait()

    return k(idx, table_scaled)


def kernel(x, table):
    b, s = x.shape
    n = b * s
    v = table.shape[0]
    xf = x.reshape(n).astype(jnp.int32)
    blk = xf >> 15
    m = xf & (FMT_BLK - 1)
    xg = (blk << 15) + ((m & (FMT_BLK // 2 - 1)) << 1) + (m >> 14)
    tf = _tc_format(table.T)
    th = tf.reshape(tf.shape[0] * 2, D_MODEL)
    out = _sc_embed(xg, th, n)
    return out.reshape(b, s, D_PAD)[:, :, :D_MODEL]
